# Initial kernel scaffold; baseline (speedup 1.0000x reference)
import jax
import jax.numpy as jnp
from jax.experimental import pallas as pl


def kernel(x0, x1, x2, x3, parent0, parent1, parent2, off0, off1, off2, edge_src, edge_dst, kidx, Wup0, gup0, bup0, mup0, vup0, Wup1, gup1, bup1, mup1, vup1, Wup2, gup2, bup2, mup2, vup2, Wsm, gsm, bsm, msm, vsm):
    raise NotImplementedError("write your pallas kernel here")


# speedup vs baseline: 1.3629x; 1.3629x over previous
import functools

import jax
import jax.numpy as jnp
from jax import lax
from jax.experimental import pallas as pl
from jax.experimental.pallas import tpu as pltpu
from jax.experimental.pallas import tpu_sc as plsc

EPS = 1e-5
NC, NS = 2, 16
NW = NC * NS

N0, N1, N2, N3 = 10000, 2500, 640, 160
N0P, N1P, N2G = 10240, 2560, 1024
E, EP = 160000, 163840
LD = 128
ACC_R = 10240
DUMMY_DST = 10200


def _rsqrt(v):
    return jax.lax.rsqrt(v + EPS)


def _tc_a(x3_ref, w_ref, g_ref, v_ref, o_ref):
    a = g_ref[...] * _rsqrt(v_ref[...])
    x = x3_ref[...]
    for k in range(8):
        o_ref[k] = jnp.dot(x, w_ref[k] * a, preferred_element_type=jnp.float32)


def _tc_b(up_ref, g2, b2, m2, v2, x2_ref, w_ref, g1, v1, o_ref):
    a2 = g2[...] * _rsqrt(v2[...])
    c2 = b2[...] - m2[...] * a2
    u = jnp.maximum(up_ref[0:N2] + c2, 0.0)
    r = jnp.concatenate([u, x2_ref[...]], axis=1)
    a1 = g1[...] * _rsqrt(v1[...])
    z = jnp.zeros((N2, 64), jnp.float32)
    for k in range(8):
        h = jnp.dot(r, w_ref[k] * a1, preferred_element_type=jnp.float32)
        o_ref[k] = jnp.concatenate([h, z], axis=1)


def _tc_c(up_ref, g1, b1, m1, v1, x1_ref, w_ref, g0, v0, o_ref):
    a1 = g1[...] * _rsqrt(v1[...])
    c1 = b1[...] - m1[...] * a1
    u = jnp.maximum(up_ref[:, 0:192] + c1, 0.0)
    r = jnp.concatenate([u, x1_ref[...]], axis=1)
    a0 = g0[...] * _rsqrt(v0[...])
    o_ref[0] = jnp.dot(r, w_ref[0] * a0, preferred_element_type=jnp.float32)


def _tc_d(up_ref, g0, b0, m0, v0, x0_ref, w_ref, gs, vs, o_ref):
    a0 = g0[...] * _rsqrt(v0[...])
    c0 = b0[...] - m0[...] * a0
    u = jnp.maximum(up_ref[...] + c0, 0.0)
    r = jnp.concatenate([u, x0_ref[...]], axis=1)
    asm = gs[...] * _rsqrt(vs[...])
    o_ref[0] = jnp.dot(r, w_ref[0] * asm, preferred_element_type=jnp.float32)


def _tc_e(p_ref, gs, bs, ms, vs, o_ref):
    asm = gs[...] * _rsqrt(vs[...])
    cs = bs[...] - ms[...] * asm
    s = p_ref[0] + p_ref[1]
    o_ref[...] = jnp.maximum(s[0:N0] + cs, 0.0)


def _make_gather(T, D, NTOT, MULT, CH):
    NB = NTOT // NW
    nchunks = NB // CH
    mesh = plsc.VectorSubcoreMesh(core_axis_name="c", subcore_axis_name="s")

    @functools.partial(
        pl.kernel,
        out_type=jax.ShapeDtypeStruct((NTOT, D), jnp.float32),
        mesh=mesh,
        scratch_types=[
            pltpu.VMEM((NB,), jnp.int32),
            pltpu.VMEM((NB,), jnp.int32),
            pltpu.VMEM((nchunks, CH), jnp.int32),
            pltpu.VMEM((CH, D), jnp.float32),
            pltpu.SemaphoreType.DMA,
        ],
    )
    def g(table, parent, off, out, par_v, off_v, idx_v, rows_v, sem):
        wid = lax.axis_index("s") * NC + lax.axis_index("c")
        base = wid * NB
        pltpu.sync_copy(parent.at[pl.ds(base, NB)], par_v)
        pltpu.sync_copy(off.at[pl.ds(base, NB)], off_v)
        for j in range(nchunks):
            for t in range(CH // 16):
                s0 = j * CH + t * 16
                idx_v[j, pl.ds(t * 16, 16)] = (
                    off_v[pl.ds(s0, 16)] * MULT + par_v[pl.ds(s0, 16)])
            pltpu.async_copy(table.at[idx_v.at[j]], rows_v, sem).wait()
            pltpu.sync_copy(rows_v, out.at[pl.ds(base + j * CH, CH)])

    return g


_CH_SM = 64
_NCH_SM = (EP // NW) // _CH_SM
_ZCH = 128


def _sc_smooth(table, src2, kidx2, dst2, zrows):
    mesh = plsc.VectorSubcoreMesh(core_axis_name="c", subcore_axis_name="s")
    rows_per_tile = ACC_R // NS

    @functools.partial(
        pl.kernel,
        out_type=jax.ShapeDtypeStruct((NC, ACC_R, LD), jnp.float32),
        mesh=mesh,
        scratch_types=[
            pltpu.VMEM((_NCH_SM, _CH_SM), jnp.int32),
            pltpu.VMEM((_NCH_SM, _CH_SM), jnp.int32),
            pltpu.VMEM((_NCH_SM, _CH_SM), jnp.int32),
            pltpu.VMEM((_CH_SM, LD), jnp.float32),
            pltpu.VMEM((_CH_SM, LD), jnp.float32),
            pltpu.VMEM_SHARED((ACC_R, LD), jnp.float32),
            pltpu.SemaphoreType.DMA,
            pltpu.SemaphoreType.DMA,
        ],
    )
    def k(tab, src, kidx, dst, zr, out,
          gix_v, kid_v, dix_v, rows0, rows1, acc, sem0, sem1):
        cid = lax.axis_index("c")
        sid = lax.axis_index("s")
        wid = sid * NC + cid
        for j in range(rows_per_tile // _ZCH):
            pltpu.sync_copy(zr, acc.at[pl.ds(sid * rows_per_tile + j * _ZCH,
                                             _ZCH)])
        pltpu.sync_copy(src.at[pl.ds(wid * _NCH_SM, _NCH_SM)], gix_v)
        pltpu.sync_copy(kidx.at[pl.ds(wid * _NCH_SM, _NCH_SM)], kid_v)
        pltpu.sync_copy(dst.at[pl.ds(wid * _NCH_SM, _NCH_SM)], dix_v)
        def ibody(j, carry):
            for t in range(_CH_SM // 16):
                sl = pl.ds(t * 16, 16)
                gix_v[j, sl] = kid_v[j, sl] * N0P + gix_v[j, sl]
            return carry

        lax.fori_loop(0, _NCH_SM, ibody, 0)
        plsc.subcore_barrier()

        def gbody(i, carry):
            j0 = i * 2
            pltpu.async_copy(tab.at[gix_v.at[j0]], rows0, sem0)
            pltpu.async_copy(tab.at[gix_v.at[j0 + 1]], rows1, sem1)
            pltpu.make_async_copy(tab.at[gix_v.at[j0]], rows0, sem0).wait()
            pltpu.sync_copy(rows0, acc.at[dix_v.at[j0]], add=True)
            pltpu.make_async_copy(tab.at[gix_v.at[j0 + 1]], rows1,
                                  sem1).wait()
            pltpu.sync_copy(rows1, acc.at[dix_v.at[j0 + 1]], add=True)
            return carry

        lax.fori_loop(0, _NCH_SM // 2, gbody, 0)
        plsc.subcore_barrier()
        pltpu.sync_copy(acc.at[pl.ds(sid * rows_per_tile, rows_per_tile)],
                        out.at[cid, pl.ds(sid * rows_per_tile, rows_per_tile)])

    return k(table, src2, kidx2, dst2, zrows)


def kernel(x0, x1, x2, x3, parent0, parent1, parent2, off0, off1, off2,
           edge_src, edge_dst, kidx,
           Wup0, gup0, bup0, mup0, vup0, Wup1, gup1, bup1, mup1, vup1,
           Wup2, gup2, bup2, mup2, vup2, Wsm, gsm, bsm, msm, vsm):
    r2 = lambda p: p.reshape(1, -1)
    g0r, b0r, m0r, v0r = r2(gup0), r2(bup0), r2(mup0), r2(vup0)
    g1r, b1r, m1r, v1r = r2(gup1), r2(bup1), r2(mup1), r2(vup1)
    g2r, b2r, m2r, v2r = r2(gup2), r2(bup2), r2(mup2), r2(vup2)
    gsr, bsr, msr, vsr = r2(gsm), r2(bsm), r2(msm), r2(vsm)

    p2p = jnp.pad(parent2, (0, N2G - N2))
    o2p = jnp.pad(off2, (0, N2G - N2))
    p1p = jnp.pad(parent1, (0, N1P - N1))
    o1p = jnp.pad(off1, (0, N1P - N1))
    p0p = jnp.pad(parent0, (0, N0P - N0))
    o0p = jnp.pad(off0, (0, N0P - N0))
    x1p = jnp.pad(x1, ((0, N1P - N1), (0, 0)))
    x0p = jnp.pad(x0, ((0, N0P - N0), (0, 0)))
    src2 = jnp.pad(edge_src, (0, EP - E)).reshape(NW * _NCH_SM, _CH_SM)
    kid2 = jnp.pad(kidx, (0, EP - E)).reshape(NW * _NCH_SM, _CH_SM)
    dst2 = jnp.pad(edge_dst, (0, EP - E),
                   constant_values=DUMMY_DST).reshape(NW * _NCH_SM, _CH_SM)
    zrows = jnp.zeros((_ZCH, LD), jnp.float32)

    h2 = pl.pallas_call(
        _tc_a,
        out_shape=jax.ShapeDtypeStruct((8, N3, 128), jnp.float32),
    )(x3, Wup2, g2r, v2r)
    up2 = _make_gather(8 * N3, 128, N2G, N3, 32)(h2.reshape(8 * N3, 128),
                                                 p2p, o2p)

    h1 = pl.pallas_call(
        _tc_b,
        out_shape=jax.ShapeDtypeStruct((8, N2, 256), jnp.float32),
    )(up2, g2r, b2r, m2r, v2r, x2, Wup1, g1r, v1r)
    up1 = _make_gather(8 * N2, 256, N1P, N2, 80)(h1.reshape(8 * N2, 256),
                                                 p1p, o1p)

    h0 = pl.pallas_call(
        _tc_c,
        grid=(8,),
        in_specs=[
            pl.BlockSpec((N1P, 256), lambda k: (0, 0)),
            pl.BlockSpec((1, 192), lambda k: (0, 0)),
            pl.BlockSpec((1, 192), lambda k: (0, 0)),
            pl.BlockSpec((1, 192), lambda k: (0, 0)),
            pl.BlockSpec((1, 192), lambda k: (0, 0)),
            pl.BlockSpec((N1P, 64), lambda k: (0, 0)),
            pl.BlockSpec((1, 256, 256), lambda k: (k, 0, 0)),
            pl.BlockSpec((1, 256), lambda k: (0, 0)),
            pl.BlockSpec((1, 256), lambda k: (0, 0)),
        ],
        out_specs=pl.BlockSpec((1, N1P, 256), lambda k: (k, 0, 0)),
        out_shape=jax.ShapeDtypeStruct((8, N1P, 256), jnp.float32),
    )(up1, g1r, b1r, m1r, v1r, x1p, Wup0, g0r, v0r)
    up0 = _make_gather(8 * N1P, 256, N0P, N1P, 80)(h0.reshape(8 * N1P, 256),
                                                   p0p, o0p)

    TB = 2048
    hs = pl.pallas_call(
        _tc_d,
        grid=(N0P // TB, 27),
        in_specs=[
            pl.BlockSpec((TB, 256), lambda t, k: (t, 0)),
            pl.BlockSpec((1, 256), lambda t, k: (0, 0)),
            pl.BlockSpec((1, 256), lambda t, k: (0, 0)),
            pl.BlockSpec((1, 256), lambda t, k: (0, 0)),
            pl.BlockSpec((1, 256), lambda t, k: (0, 0)),
            pl.BlockSpec((TB, 64), lambda t, k: (t, 0)),
            pl.BlockSpec((1, 320, 128), lambda t, k: (k, 0, 0)),
            pl.BlockSpec((1, 128), lambda t, k: (0, 0)),
            pl.BlockSpec((1, 128), lambda t, k: (0, 0)),
        ],
        out_specs=pl.BlockSpec((1, TB, 128), lambda t, k: (k, t, 0)),
        out_shape=jax.ShapeDtypeStruct((27, N0P, 128), jnp.float32),
    )(up0, g0r, b0r, m0r, v0r, x0p, Wsm, gsr, vsr)

    parts = _sc_smooth(hs.reshape(27 * N0P, 128), src2, kid2, dst2, zrows)

    out = pl.pallas_call(
        _tc_e,
        out_shape=jax.ShapeDtypeStruct((N0, LD), jnp.float32),
    )(parts, gsr, bsr, msr, vsr)
    return out


# speedup vs baseline: 1.5241x; 1.1183x over previous
import functools

import jax
import jax.numpy as jnp
from jax import lax
from jax.experimental import pallas as pl
from jax.experimental.pallas import tpu as pltpu
from jax.experimental.pallas import tpu_sc as plsc

EPS = 1e-5
NC, NS = 2, 16
NW = NC * NS

N0, N1, N2, N3 = 10000, 2500, 640, 160
N0P, N1P, N2G = 10240, 2560, 1024
E, EP = 160000, 163840
LD = 128
ACC_R = 10240
DUMMY_DST = 10200


def _rsqrt(v):
    return jax.lax.rsqrt(v + EPS)


def _tc_a(x3_ref, w_ref, g_ref, v_ref, o_ref):
    a = g_ref[...] * _rsqrt(v_ref[...])
    x = x3_ref[...]
    for k in range(8):
        o_ref[k] = jnp.dot(x, w_ref[k] * a, preferred_element_type=jnp.float32)


def _tc_b(up_ref, g2, b2, m2, v2, x2_ref, w_ref, g1, v1, o_ref):
    a2 = g2[...] * _rsqrt(v2[...])
    c2 = b2[...] - m2[...] * a2
    u = jnp.maximum(up_ref[0:N2] + c2, 0.0)
    r = jnp.concatenate([u, x2_ref[...]], axis=1)
    a1 = g1[...] * _rsqrt(v1[...])
    z = jnp.zeros((N2, 64), jnp.float32)
    for k in range(8):
        h = jnp.dot(r, w_ref[k] * a1, preferred_element_type=jnp.float32)
        o_ref[k] = jnp.concatenate([h, z], axis=1)


def _tc_c(up_ref, g1, b1, m1, v1, x1_ref, w_ref, g0, v0, o_ref):
    a1 = g1[...] * _rsqrt(v1[...])
    c1 = b1[...] - m1[...] * a1
    u = jnp.maximum(up_ref[:, 0:192] + c1, 0.0)
    r = jnp.concatenate([u, x1_ref[...]], axis=1)
    a0 = g0[...] * _rsqrt(v0[...])
    o_ref[0] = jnp.dot(r, w_ref[0] * a0, preferred_element_type=jnp.float32)


def _tc_d(up_ref, g0, b0, m0, v0, x0_ref, w_ref, gs, vs, o_ref):
    a0 = g0[...] * _rsqrt(v0[...])
    c0 = b0[...] - m0[...] * a0
    u = jnp.maximum(up_ref[...] + c0, 0.0)
    r = jnp.concatenate([u, x0_ref[...]], axis=1)
    asm = gs[...] * _rsqrt(vs[...])
    o_ref[0] = jnp.dot(r, w_ref[0] * asm, preferred_element_type=jnp.float32)


def _tc_e(p_ref, gs, bs, ms, vs, o_ref):
    asm = gs[...] * _rsqrt(vs[...])
    cs = bs[...] - ms[...] * asm
    s = p_ref[0] + p_ref[1]
    o_ref[...] = jnp.maximum(s[0:N0] + cs, 0.0)


def _make_gather(T, D, NTOT, MULT, CH):
    NB = NTOT // NW
    nchunks = NB // CH
    mesh = plsc.VectorSubcoreMesh(core_axis_name="c", subcore_axis_name="s")

    @functools.partial(
        pl.kernel,
        out_type=jax.ShapeDtypeStruct((NTOT, D), jnp.float32),
        mesh=mesh,
        scratch_types=[
            pltpu.VMEM((NB,), jnp.int32),
            pltpu.VMEM((NB,), jnp.int32),
            pltpu.VMEM((nchunks, CH), jnp.int32),
            pltpu.VMEM((CH, D), jnp.float32),
            pltpu.SemaphoreType.DMA,
        ],
    )
    def g(table, parent, off, out, par_v, off_v, idx_v, rows_v, sem):
        wid = lax.axis_index("s") * NC + lax.axis_index("c")
        base = wid * NB
        pltpu.sync_copy(parent.at[pl.ds(base, NB)], par_v)
        pltpu.sync_copy(off.at[pl.ds(base, NB)], off_v)
        for j in range(nchunks):
            for t in range(CH // 16):
                s0 = j * CH + t * 16
                idx_v[j, pl.ds(t * 16, 16)] = (
                    off_v[pl.ds(s0, 16)] * MULT + par_v[pl.ds(s0, 16)])
            pltpu.async_copy(table.at[idx_v.at[j]], rows_v, sem).wait()
            pltpu.sync_copy(rows_v, out.at[pl.ds(base + j * CH, CH)])

    return g


_CH_SM = 128
_NCH_SM = (EP // NW) // _CH_SM
_ZCH = 128


def _sc_smooth(table, src2, kidx2, dst2, zrows):
    mesh = plsc.VectorSubcoreMesh(core_axis_name="c", subcore_axis_name="s")
    rows_per_tile = ACC_R // NS

    @functools.partial(
        pl.kernel,
        out_type=jax.ShapeDtypeStruct((NC, ACC_R, LD), jnp.float32),
        mesh=mesh,
        scratch_types=[
            pltpu.VMEM((_NCH_SM, _CH_SM), jnp.int32),
            pltpu.VMEM((_NCH_SM, _CH_SM), jnp.int32),
            pltpu.VMEM((_NCH_SM, _CH_SM), jnp.int32),
            [pltpu.VMEM((_CH_SM, LD), jnp.float32) for _ in range(2)],
            [pltpu.SemaphoreType.DMA for _ in range(2)],
            [pltpu.SemaphoreType.DMA for _ in range(2)],
            pltpu.VMEM_SHARED((ACC_R, LD), jnp.float32),
            pltpu.SemaphoreType.DMA,
        ],
    )
    def k(tab, src, kidx, dst, zr, out,
          gix_v, kid_v, dix_v, bufs, gsems, ssems, acc, psem):
        cid = lax.axis_index("c")
        sid = lax.axis_index("s")
        wid = sid * NC + cid
        nz = rows_per_tile // _ZCH
        for j in range(nz):
            pltpu.async_copy(zr, acc.at[pl.ds(sid * rows_per_tile + j * _ZCH,
                                              _ZCH)], psem)
        pltpu.async_copy(src.at[pl.ds(wid * _NCH_SM, _NCH_SM)], gix_v, psem)
        pltpu.async_copy(kidx.at[pl.ds(wid * _NCH_SM, _NCH_SM)], kid_v, psem)
        pltpu.async_copy(dst.at[pl.ds(wid * _NCH_SM, _NCH_SM)], dix_v, psem)
        for j in range(nz):
            pltpu.make_async_copy(zr, acc.at[pl.ds(0, _ZCH)], psem).wait()
        pltpu.make_async_copy(src.at[pl.ds(0, _NCH_SM)], gix_v, psem).wait()
        pltpu.make_async_copy(kidx.at[pl.ds(0, _NCH_SM)], kid_v, psem).wait()
        pltpu.make_async_copy(dst.at[pl.ds(0, _NCH_SM)], dix_v, psem).wait()

        def ibody(j, carry):
            for t in range(_CH_SM // 16):
                sl = pl.ds(t * 16, 16)
                gix_v[j, sl] = kid_v[j, sl] * N0P + gix_v[j, sl]
            return carry

        lax.fori_loop(0, _NCH_SM, ibody, 0)
        plsc.subcore_barrier()

        for t in range(2):
            pltpu.async_copy(tab.at[gix_v.at[t]], bufs[t], gsems[t])

        def gbody(i, carry):
            j = i * 2
            for t in range(2):
                c = j + t
                pltpu.make_async_copy(tab.at[gix_v.at[c]], bufs[t],
                                      gsems[t]).wait()
                pltpu.async_copy(bufs[t], acc.at[dix_v.at[c]], ssems[t],
                                 add=True)
            for t in range(2):
                c = j + t
                pltpu.make_async_copy(bufs[t], acc.at[dix_v.at[c]],
                                      ssems[t]).wait()

                @pl.when(c + 2 < _NCH_SM)
                def _():
                    pltpu.async_copy(tab.at[gix_v.at[c + 2]], bufs[t],
                                     gsems[t])
            return carry

        lax.fori_loop(0, _NCH_SM // 2, gbody, 0)
        plsc.subcore_barrier()
        pltpu.sync_copy(acc.at[pl.ds(sid * rows_per_tile, rows_per_tile)],
                        out.at[cid, pl.ds(sid * rows_per_tile, rows_per_tile)])

    return k(table, src2, kidx2, dst2, zrows)


def kernel(x0, x1, x2, x3, parent0, parent1, parent2, off0, off1, off2,
           edge_src, edge_dst, kidx,
           Wup0, gup0, bup0, mup0, vup0, Wup1, gup1, bup1, mup1, vup1,
           Wup2, gup2, bup2, mup2, vup2, Wsm, gsm, bsm, msm, vsm):
    r2 = lambda p: p.reshape(1, -1)
    g0r, b0r, m0r, v0r = r2(gup0), r2(bup0), r2(mup0), r2(vup0)
    g1r, b1r, m1r, v1r = r2(gup1), r2(bup1), r2(mup1), r2(vup1)
    g2r, b2r, m2r, v2r = r2(gup2), r2(bup2), r2(mup2), r2(vup2)
    gsr, bsr, msr, vsr = r2(gsm), r2(bsm), r2(msm), r2(vsm)

    p2p = jnp.pad(parent2, (0, N2G - N2))
    o2p = jnp.pad(off2, (0, N2G - N2))
    p1p = jnp.pad(parent1, (0, N1P - N1))
    o1p = jnp.pad(off1, (0, N1P - N1))
    p0p = jnp.pad(parent0, (0, N0P - N0))
    o0p = jnp.pad(off0, (0, N0P - N0))
    x1p = jnp.pad(x1, ((0, N1P - N1), (0, 0)))
    x0p = jnp.pad(x0, ((0, N0P - N0), (0, 0)))
    src2 = jnp.pad(edge_src, (0, EP - E)).reshape(NW * _NCH_SM, _CH_SM)
    kid2 = jnp.pad(kidx, (0, EP - E)).reshape(NW * _NCH_SM, _CH_SM)
    dst2 = jnp.pad(edge_dst, (0, EP - E),
                   constant_values=DUMMY_DST).reshape(NW * _NCH_SM, _CH_SM)
    zrows = jnp.zeros((_ZCH, LD), jnp.float32)

    h2 = pl.pallas_call(
        _tc_a,
        out_shape=jax.ShapeDtypeStruct((8, N3, 128), jnp.float32),
    )(x3, Wup2, g2r, v2r)
    up2 = _make_gather(8 * N3, 128, N2G, N3, 32)(h2.reshape(8 * N3, 128),
                                                 p2p, o2p)

    h1 = pl.pallas_call(
        _tc_b,
        out_shape=jax.ShapeDtypeStruct((8, N2, 256), jnp.float32),
    )(up2, g2r, b2r, m2r, v2r, x2, Wup1, g1r, v1r)
    up1 = _make_gather(8 * N2, 256, N1P, N2, 80)(h1.reshape(8 * N2, 256),
                                                 p1p, o1p)

    h0 = pl.pallas_call(
        _tc_c,
        grid=(8,),
        in_specs=[
            pl.BlockSpec((N1P, 256), lambda k: (0, 0)),
            pl.BlockSpec((1, 192), lambda k: (0, 0)),
            pl.BlockSpec((1, 192), lambda k: (0, 0)),
            pl.BlockSpec((1, 192), lambda k: (0, 0)),
            pl.BlockSpec((1, 192), lambda k: (0, 0)),
            pl.BlockSpec((N1P, 64), lambda k: (0, 0)),
            pl.BlockSpec((1, 256, 256), lambda k: (k, 0, 0)),
            pl.BlockSpec((1, 256), lambda k: (0, 0)),
            pl.BlockSpec((1, 256), lambda k: (0, 0)),
        ],
        out_specs=pl.BlockSpec((1, N1P, 256), lambda k: (k, 0, 0)),
        out_shape=jax.ShapeDtypeStruct((8, N1P, 256), jnp.float32),
    )(up1, g1r, b1r, m1r, v1r, x1p, Wup0, g0r, v0r)
    up0 = _make_gather(8 * N1P, 256, N0P, N1P, 80)(h0.reshape(8 * N1P, 256),
                                                   p0p, o0p)

    TB = 2048
    hs = pl.pallas_call(
        _tc_d,
        grid=(N0P // TB, 27),
        in_specs=[
            pl.BlockSpec((TB, 256), lambda t, k: (t, 0)),
            pl.BlockSpec((1, 256), lambda t, k: (0, 0)),
            pl.BlockSpec((1, 256), lambda t, k: (0, 0)),
            pl.BlockSpec((1, 256), lambda t, k: (0, 0)),
            pl.BlockSpec((1, 256), lambda t, k: (0, 0)),
            pl.BlockSpec((TB, 64), lambda t, k: (t, 0)),
            pl.BlockSpec((1, 320, 128), lambda t, k: (k, 0, 0)),
            pl.BlockSpec((1, 128), lambda t, k: (0, 0)),
            pl.BlockSpec((1, 128), lambda t, k: (0, 0)),
        ],
        out_specs=pl.BlockSpec((1, TB, 128), lambda t, k: (k, t, 0)),
        out_shape=jax.ShapeDtypeStruct((27, N0P, 128), jnp.float32),
    )(up0, g0r, b0r, m0r, v0r, x0p, Wsm, gsr, vsr)

    parts = _sc_smooth(hs.reshape(27 * N0P, 128), src2, kid2, dst2, zrows)

    out = pl.pallas_call(
        _tc_e,
        out_shape=jax.ShapeDtypeStruct((N0, LD), jnp.float32),
    )(parts, gsr, bsr, msr, vsr)
    return out


# speedup vs baseline: 1.6830x; 1.1042x over previous
import functools

import jax
import jax.numpy as jnp
from jax import lax
from jax.experimental import pallas as pl
from jax.experimental.pallas import tpu as pltpu
from jax.experimental.pallas import tpu_sc as plsc

EPS = 1e-5
NC, NS = 2, 16
NW = NC * NS

N0, N1, N2, N3 = 10000, 2500, 640, 160
N0P, N1P, N2G = 10240, 2560, 1024
E, EP = 160000, 163840
LD = 128
ACC_R = 10240
DUMMY_DST = 10200


def _rsqrt(v):
    return jax.lax.rsqrt(v + EPS)


def _tc_a(x3_ref, w_ref, g_ref, v_ref, o_ref):
    a = g_ref[...] * _rsqrt(v_ref[...])
    x = x3_ref[...]
    for k in range(8):
        o_ref[k] = jnp.dot(x, w_ref[k] * a, preferred_element_type=jnp.float32)


def _tc_b(up_ref, g2, b2, m2, v2, x2_ref, w_ref, g1, v1, o_ref):
    a2 = g2[...] * _rsqrt(v2[...])
    c2 = b2[...] - m2[...] * a2
    u = jnp.maximum(up_ref[0:N2] + c2, 0.0)
    r = jnp.concatenate([u, x2_ref[...]], axis=1)
    a1 = g1[...] * _rsqrt(v1[...])
    z = jnp.zeros((N2, 64), jnp.float32)
    for k in range(8):
        h = jnp.dot(r, w_ref[k] * a1, preferred_element_type=jnp.float32)
        o_ref[k] = jnp.concatenate([h, z], axis=1)


def _tc_c(up_ref, g1, b1, m1, v1, x1_ref, w_ref, g0, v0, o_ref):
    a1 = g1[...] * _rsqrt(v1[...])
    c1 = b1[...] - m1[...] * a1
    u = jnp.maximum(up_ref[:, 0:192] + c1, 0.0)
    r = jnp.concatenate([u, x1_ref[...]], axis=1)
    a0 = g0[...] * _rsqrt(v0[...])
    o_ref[0] = jnp.dot(r, w_ref[0] * a0, preferred_element_type=jnp.float32)


def _tc_d(up_ref, g0, b0, m0, v0, x0_ref, w_ref, gs, vs, o_ref):
    a0 = g0[...] * _rsqrt(v0[...])
    c0 = b0[...] - m0[...] * a0
    u = jnp.maximum(up_ref[...] + c0, 0.0)
    r = jnp.concatenate([u, x0_ref[...]], axis=1)
    asm = gs[...] * _rsqrt(vs[...])
    o_ref[0] = jnp.dot(r, w_ref[0] * asm, preferred_element_type=jnp.float32)


def _tc_e(p_ref, gs, bs, ms, vs, o_ref):
    asm = gs[...] * _rsqrt(vs[...])
    cs = bs[...] - ms[...] * asm
    s = p_ref[0] + p_ref[1]
    o_ref[...] = jnp.maximum(s[0:N0] + cs, 0.0)


def _make_gather(T, D, NTOT, MULT, CH):
    NB = NTOT // NW
    nchunks = NB // CH
    mesh = plsc.VectorSubcoreMesh(core_axis_name="c", subcore_axis_name="s")

    @functools.partial(
        pl.kernel,
        out_type=jax.ShapeDtypeStruct((NTOT, D), jnp.float32),
        mesh=mesh,
        scratch_types=[
            pltpu.VMEM((NB,), jnp.int32),
            pltpu.VMEM((NB,), jnp.int32),
            pltpu.VMEM((nchunks, CH), jnp.int32),
            pltpu.VMEM((CH, D), jnp.float32),
            pltpu.SemaphoreType.DMA,
        ],
    )
    def g(table, parent, off, out, par_v, off_v, idx_v, rows_v, sem):
        wid = lax.axis_index("s") * NC + lax.axis_index("c")
        base = wid * NB
        pltpu.sync_copy(parent.at[pl.ds(base, NB)], par_v)
        pltpu.sync_copy(off.at[pl.ds(base, NB)], off_v)
        for j in range(nchunks):
            for t in range(CH // 16):
                s0 = j * CH + t * 16
                idx_v[j, pl.ds(t * 16, 16)] = (
                    off_v[pl.ds(s0, 16)] * MULT + par_v[pl.ds(s0, 16)])
            pltpu.async_copy(table.at[idx_v.at[j]], rows_v, sem).wait()
            pltpu.sync_copy(rows_v, out.at[pl.ds(base + j * CH, CH)])

    return g


_CH_SM = 128
_NCHT = EP // _CH_SM
_CH_A = 60
_CH_B = _NCHT // NS - _CH_A
_ZCH = 128


def _sc_smooth(table, esk, zrows):
    mesh = plsc.VectorSubcoreMesh(core_axis_name="c", subcore_axis_name="s")
    rows_per_tile = ACC_R // NS

    @functools.partial(
        pl.kernel,
        out_type=jax.ShapeDtypeStruct((NC, ACC_R, LD), jnp.float32),
        mesh=mesh,
        scratch_types=[
            [pltpu.VMEM((1, 3, _CH_SM), jnp.int32) for _ in range(2)],
            [pltpu.VMEM((1, _CH_SM), jnp.int32) for _ in range(2)],
            [pltpu.VMEM((1, _CH_SM), jnp.int32) for _ in range(2)],
            [pltpu.VMEM((_CH_SM, LD), jnp.float32) for _ in range(2)],
            [pltpu.SemaphoreType.DMA for _ in range(2)],
            [pltpu.SemaphoreType.DMA for _ in range(2)],
            [pltpu.SemaphoreType.DMA for _ in range(2)],
            pltpu.VMEM_SHARED((ACC_R, LD), jnp.float32),
            pltpu.SemaphoreType.DMA,
        ],
    )
    def k(tab, ed, zr, out, ebufs, gixs, dixs, rows, isems, gsems, ssems,
          acc, psem):
        cid = lax.axis_index("c")
        sid = lax.axis_index("s")
        start = jnp.where(cid == 0, sid * _CH_A, NS * _CH_A + sid * _CH_B)
        n2 = jnp.where(cid == 0, _CH_A // 2, _CH_B // 2)

        nz = rows_per_tile // _ZCH
        for j in range(nz):
            pltpu.async_copy(zr, acc.at[pl.ds(sid * rows_per_tile + j * _ZCH,
                                              _ZCH)], psem)
        for t in range(2):
            pltpu.async_copy(ed.at[pl.ds(start + t, 1)], ebufs[t], isems[t])
        for j in range(nz):
            pltpu.make_async_copy(zr, acc.at[pl.ds(0, _ZCH)], psem).wait()
        plsc.subcore_barrier()

        def gbody(i, carry):
            for t in range(2):
                c = start + i * 2 + t
                pltpu.make_async_copy(ed.at[pl.ds(c, 1)], ebufs[t],
                                      isems[t]).wait()

                @pl.when(i > 0)
                def _():
                    pltpu.make_async_copy(rows[t], acc.at[dixs[t].at[0]],
                                          ssems[t]).wait()

                for v in range(_CH_SM // 16):
                    sl = pl.ds(v * 16, 16)
                    gixs[t][0, sl] = (ebufs[t][0, 1, sl] * N0P
                                      + ebufs[t][0, 0, sl])
                    dixs[t][0, sl] = ebufs[t][0, 2, sl]
                pltpu.async_copy(tab.at[gixs[t].at[0]], rows[t], gsems[t])
            for t in range(2):
                c = start + i * 2 + t
                pltpu.make_async_copy(tab.at[gixs[t].at[0]], rows[t],
                                      gsems[t]).wait()
                pltpu.async_copy(rows[t], acc.at[dixs[t].at[0]], ssems[t],
                                 add=True)

                @pl.when(i + 1 < n2)
                def _():
                    pltpu.async_copy(ed.at[pl.ds(c + 2, 1)], ebufs[t],
                                     isems[t])
            return carry

        lax.fori_loop(0, n2, gbody, 0)
        for t in range(2):
            pltpu.make_async_copy(rows[t], acc.at[dixs[t].at[0]],
                                  ssems[t]).wait()
        plsc.subcore_barrier()
        pltpu.sync_copy(acc.at[pl.ds(sid * rows_per_tile, rows_per_tile)],
                        out.at[cid, pl.ds(sid * rows_per_tile, rows_per_tile)])

    return k(table, esk, zrows)


def kernel(x0, x1, x2, x3, parent0, parent1, parent2, off0, off1, off2,
           edge_src, edge_dst, kidx,
           Wup0, gup0, bup0, mup0, vup0, Wup1, gup1, bup1, mup1, vup1,
           Wup2, gup2, bup2, mup2, vup2, Wsm, gsm, bsm, msm, vsm):
    r2 = lambda p: p.reshape(1, -1)
    g0r, b0r, m0r, v0r = r2(gup0), r2(bup0), r2(mup0), r2(vup0)
    g1r, b1r, m1r, v1r = r2(gup1), r2(bup1), r2(mup1), r2(vup1)
    g2r, b2r, m2r, v2r = r2(gup2), r2(bup2), r2(mup2), r2(vup2)
    gsr, bsr, msr, vsr = r2(gsm), r2(bsm), r2(msm), r2(vsm)

    p2p = jnp.pad(parent2, (0, N2G - N2))
    o2p = jnp.pad(off2, (0, N2G - N2))
    p1p = jnp.pad(parent1, (0, N1P - N1))
    o1p = jnp.pad(off1, (0, N1P - N1))
    p0p = jnp.pad(parent0, (0, N0P - N0))
    o0p = jnp.pad(off0, (0, N0P - N0))
    x1p = jnp.pad(x1, ((0, N1P - N1), (0, 0)))
    x0p = jnp.pad(x0, ((0, N0P - N0), (0, 0)))
    src2 = jnp.pad(edge_src, (0, EP - E)).reshape(_NCHT, _CH_SM)
    kid2 = jnp.pad(kidx, (0, EP - E)).reshape(_NCHT, _CH_SM)
    dst2 = jnp.pad(edge_dst, (0, EP - E),
                   constant_values=DUMMY_DST).reshape(_NCHT, _CH_SM)
    esk = jnp.stack([src2, kid2, dst2], axis=1)
    zrows = jnp.zeros((_ZCH, LD), jnp.float32)

    h2 = pl.pallas_call(
        _tc_a,
        out_shape=jax.ShapeDtypeStruct((8, N3, 128), jnp.float32),
    )(x3, Wup2, g2r, v2r)
    up2 = _make_gather(8 * N3, 128, N2G, N3, 32)(h2.reshape(8 * N3, 128),
                                                 p2p, o2p)

    h1 = pl.pallas_call(
        _tc_b,
        out_shape=jax.ShapeDtypeStruct((8, N2, 256), jnp.float32),
    )(up2, g2r, b2r, m2r, v2r, x2, Wup1, g1r, v1r)
    up1 = _make_gather(8 * N2, 256, N1P, N2, 80)(h1.reshape(8 * N2, 256),
                                                 p1p, o1p)

    h0 = pl.pallas_call(
        _tc_c,
        grid=(8,),
        in_specs=[
            pl.BlockSpec((N1P, 256), lambda k: (0, 0)),
            pl.BlockSpec((1, 192), lambda k: (0, 0)),
            pl.BlockSpec((1, 192), lambda k: (0, 0)),
            pl.BlockSpec((1, 192), lambda k: (0, 0)),
            pl.BlockSpec((1, 192), lambda k: (0, 0)),
            pl.BlockSpec((N1P, 64), lambda k: (0, 0)),
            pl.BlockSpec((1, 256, 256), lambda k: (k, 0, 0)),
            pl.BlockSpec((1, 256), lambda k: (0, 0)),
            pl.BlockSpec((1, 256), lambda k: (0, 0)),
        ],
        out_specs=pl.BlockSpec((1, N1P, 256), lambda k: (k, 0, 0)),
        out_shape=jax.ShapeDtypeStruct((8, N1P, 256), jnp.float32),
    )(up1, g1r, b1r, m1r, v1r, x1p, Wup0, g0r, v0r)
    up0 = _make_gather(8 * N1P, 256, N0P, N1P, 80)(h0.reshape(8 * N1P, 256),
                                                   p0p, o0p)

    TB = 2048
    hs = pl.pallas_call(
        _tc_d,
        grid=(N0P // TB, 27),
        in_specs=[
            pl.BlockSpec((TB, 256), lambda t, k: (t, 0)),
            pl.BlockSpec((1, 256), lambda t, k: (0, 0)),
            pl.BlockSpec((1, 256), lambda t, k: (0, 0)),
            pl.BlockSpec((1, 256), lambda t, k: (0, 0)),
            pl.BlockSpec((1, 256), lambda t, k: (0, 0)),
            pl.BlockSpec((TB, 64), lambda t, k: (t, 0)),
            pl.BlockSpec((1, 320, 128), lambda t, k: (k, 0, 0)),
            pl.BlockSpec((1, 128), lambda t, k: (0, 0)),
            pl.BlockSpec((1, 128), lambda t, k: (0, 0)),
        ],
        out_specs=pl.BlockSpec((1, TB, 128), lambda t, k: (k, t, 0)),
        out_shape=jax.ShapeDtypeStruct((27, N0P, 128), jnp.float32),
    )(up0, g0r, b0r, m0r, v0r, x0p, Wsm, gsr, vsr)

    parts = _sc_smooth(hs.reshape(27 * N0P, 128), esk, zrows)

    out = pl.pallas_call(
        _tc_e,
        out_shape=jax.ShapeDtypeStruct((N0, LD), jnp.float32),
    )(parts, gsr, bsr, msr, vsr)
    return out


# speedup vs baseline: 1.7263x; 1.0257x over previous
import functools

import jax
import jax.numpy as jnp
from jax import lax
from jax.experimental import pallas as pl
from jax.experimental.pallas import tpu as pltpu
from jax.experimental.pallas import tpu_sc as plsc

EPS = 1e-5
NC, NS = 2, 16
NW = NC * NS

N0, N1, N2, N3 = 10000, 2500, 640, 160
N0P, N1P, N2G = 10240, 2560, 1024
E, EP = 160000, 163840
LD = 128
ACC_R = 10240
DUMMY_DST = 10200


def _rsqrt(v):
    return jax.lax.rsqrt(v + EPS)


def _tc_a(x3_ref, w_ref, g_ref, v_ref, o_ref):
    a = g_ref[...] * _rsqrt(v_ref[...])
    x = x3_ref[...]
    for k in range(8):
        o_ref[k] = jnp.dot(x, w_ref[k] * a, preferred_element_type=jnp.float32)


def _tc_b(p2_ref, o2_ref, h2_ref, g2, b2, m2, v2, x2_ref, w_ref, g1, v1,
          o_ref):
    flat = o2_ref[0] * N3 + p2_ref[0]
    oh = jnp.where(
        jax.lax.broadcasted_iota(jnp.int32, (N2, 8 * N3), 1)
        == flat.reshape(N2, 1), 1.0, 0.0).astype(jnp.float32)
    up2 = jnp.dot(oh, h2_ref[...], preferred_element_type=jnp.float32)
    a2 = g2[...] * _rsqrt(v2[...])
    c2 = b2[...] - m2[...] * a2
    u = jnp.maximum(up2 + c2, 0.0)
    r = jnp.concatenate([u, x2_ref[...]], axis=1)
    a1 = g1[...] * _rsqrt(v1[...])
    z = jnp.zeros((N2, 64), jnp.float32)
    for k in range(8):
        h = jnp.dot(r, w_ref[k] * a1, preferred_element_type=jnp.float32)
        o_ref[k] = jnp.concatenate([h, z], axis=1)


def _tc_c(up_ref, g1, b1, m1, v1, x1_ref, w_ref, g0, v0, o_ref):
    a1 = g1[...] * _rsqrt(v1[...])
    c1 = b1[...] - m1[...] * a1
    u = jnp.maximum(up_ref[:, 0:192] + c1, 0.0)
    r = jnp.concatenate([u, x1_ref[...]], axis=1).astype(jnp.bfloat16)
    a0 = g0[...] * _rsqrt(v0[...])
    w = (w_ref[0] * a0).astype(jnp.bfloat16)
    o_ref[0] = jnp.dot(r, w, preferred_element_type=jnp.float32)


def _tc_d(up_ref, g0, b0, m0, v0, x0_ref, w_ref, gs, vs, o_ref):
    a0 = g0[...] * _rsqrt(v0[...])
    c0 = b0[...] - m0[...] * a0
    u = jnp.maximum(up_ref[...] + c0, 0.0)
    r = jnp.concatenate([u, x0_ref[...]], axis=1).astype(jnp.bfloat16)
    asm = gs[...] * _rsqrt(vs[...])
    w = (w_ref[0] * asm).astype(jnp.bfloat16)
    o_ref[0] = jnp.dot(r, w, preferred_element_type=jnp.float32)


def _tc_e(p_ref, gs, bs, ms, vs, o_ref):
    asm = gs[...] * _rsqrt(vs[...])
    cs = bs[...] - ms[...] * asm
    s = p_ref[0] + p_ref[1]
    o_ref[...] = jnp.maximum(s[0:N0] + cs, 0.0)


def _make_gather(T, D, MULT, CH, ncha, nchb):
    NTOT = (ncha + nchb) * NS * CH
    nmax = max(ncha, nchb)
    mesh = plsc.VectorSubcoreMesh(core_axis_name="c", subcore_axis_name="s")

    @functools.partial(
        pl.kernel,
        out_type=jax.ShapeDtypeStruct((NTOT, D), jnp.float32),
        mesh=mesh,
        scratch_types=[
            pltpu.VMEM((nmax * CH,), jnp.int32),
            pltpu.VMEM((nmax * CH,), jnp.int32),
            pltpu.VMEM((nmax, CH), jnp.int32),
            [pltpu.VMEM((CH, D), jnp.float32) for _ in range(nmax)],
            [pltpu.SemaphoreType.DMA for _ in range(nmax)],
            [pltpu.SemaphoreType.DMA for _ in range(nmax)],
        ],
    )
    def g(table, parent, off, out, par_v, off_v, idx_v, rows, gsems, osems):
        cid = lax.axis_index("c")
        sid = lax.axis_index("s")

        def run(base, nch):
            nb = nch * CH
            pltpu.sync_copy(parent.at[pl.ds(base, nb)],
                            par_v.at[pl.ds(0, nb)])
            pltpu.sync_copy(off.at[pl.ds(base, nb)], off_v.at[pl.ds(0, nb)])
            for j in range(nch):
                for t in range(CH // 16):
                    s0 = j * CH + t * 16
                    idx_v[j, pl.ds(t * 16, 16)] = (
                        off_v[pl.ds(s0, 16)] * MULT + par_v[pl.ds(s0, 16)])
                pltpu.async_copy(table.at[idx_v.at[j]], rows[j], gsems[j])
            for j in range(nch):
                pltpu.make_async_copy(table.at[idx_v.at[j]], rows[j],
                                      gsems[j]).wait()
                pltpu.async_copy(rows[j], out.at[pl.ds(base + j * CH, CH)],
                                 osems[j])
            for j in range(nch):
                pltpu.make_async_copy(rows[j],
                                      out.at[pl.ds(base + j * CH, CH)],
                                      osems[j]).wait()

        @pl.when(cid == 0)
        def _():
            run(sid * (ncha * CH), ncha)

        @pl.when(cid != 0)
        def _():
            run(NS * (ncha * CH) + sid * (nchb * CH), nchb)

    return g


_CH_SM = 128
_NCHT = EP // _CH_SM
_CH_A = 68
_CH_B = _NCHT // NS - _CH_A
_ZCH = 128


def _sc_smooth(table, esk, zrows):
    mesh = plsc.VectorSubcoreMesh(core_axis_name="c", subcore_axis_name="s")
    rows_per_tile = ACC_R // NS

    @functools.partial(
        pl.kernel,
        out_type=jax.ShapeDtypeStruct((NC, ACC_R, LD), jnp.float32),
        mesh=mesh,
        scratch_types=[
            [pltpu.VMEM((1, 3, _CH_SM), jnp.int32) for _ in range(2)],
            [pltpu.VMEM((1, _CH_SM), jnp.int32) for _ in range(2)],
            [pltpu.VMEM((1, _CH_SM), jnp.int32) for _ in range(2)],
            [pltpu.VMEM((_CH_SM, LD), jnp.float32) for _ in range(2)],
            [pltpu.SemaphoreType.DMA for _ in range(2)],
            [pltpu.SemaphoreType.DMA for _ in range(2)],
            [pltpu.SemaphoreType.DMA for _ in range(2)],
            pltpu.VMEM_SHARED((ACC_R, LD), jnp.float32),
            pltpu.SemaphoreType.DMA,
        ],
    )
    def k(tab, ed, zr, out, ebufs, gixs, dixs, rows, isems, gsems, ssems,
          acc, psem):
        cid = lax.axis_index("c")
        sid = lax.axis_index("s")
        start = jnp.where(cid == 0, sid * _CH_A, NS * _CH_A + sid * _CH_B)
        n2 = jnp.where(cid == 0, _CH_A // 2, _CH_B // 2)

        nz = rows_per_tile // _ZCH
        for j in range(nz):
            pltpu.async_copy(zr, acc.at[pl.ds(sid * rows_per_tile + j * _ZCH,
                                              _ZCH)], psem)
        for t in range(2):
            pltpu.async_copy(ed.at[pl.ds(start + t, 1)], ebufs[t], isems[t])
        for j in range(nz):
            pltpu.make_async_copy(zr, acc.at[pl.ds(0, _ZCH)], psem).wait()
        plsc.subcore_barrier()

        def gbody(i, carry):
            for t in range(2):
                c = start + i * 2 + t
                pltpu.make_async_copy(ed.at[pl.ds(c, 1)], ebufs[t],
                                      isems[t]).wait()

                @pl.when(i > 0)
                def _():
                    pltpu.make_async_copy(rows[t], acc.at[dixs[t].at[0]],
                                          ssems[t]).wait()

                for v in range(_CH_SM // 16):
                    sl = pl.ds(v * 16, 16)
                    gixs[t][0, sl] = (ebufs[t][0, 1, sl] * N0P
                                      + ebufs[t][0, 0, sl])
                    dixs[t][0, sl] = ebufs[t][0, 2, sl]
                pltpu.async_copy(tab.at[gixs[t].at[0]], rows[t], gsems[t])
            for t in range(2):
                c = start + i * 2 + t
                pltpu.make_async_copy(tab.at[gixs[t].at[0]], rows[t],
                                      gsems[t]).wait()
                pltpu.async_copy(rows[t], acc.at[dixs[t].at[0]], ssems[t],
                                 add=True)

                @pl.when(i + 1 < n2)
                def _():
                    pltpu.async_copy(ed.at[pl.ds(c + 2, 1)], ebufs[t],
                                     isems[t])
            return carry

        lax.fori_loop(0, n2, gbody, 0)
        for t in range(2):
            pltpu.make_async_copy(rows[t], acc.at[dixs[t].at[0]],
                                  ssems[t]).wait()
        plsc.subcore_barrier()
        pltpu.sync_copy(acc.at[pl.ds(sid * rows_per_tile, rows_per_tile)],
                        out.at[cid, pl.ds(sid * rows_per_tile, rows_per_tile)])

    return k(table, esk, zrows)


def kernel(x0, x1, x2, x3, parent0, parent1, parent2, off0, off1, off2,
           edge_src, edge_dst, kidx,
           Wup0, gup0, bup0, mup0, vup0, Wup1, gup1, bup1, mup1, vup1,
           Wup2, gup2, bup2, mup2, vup2, Wsm, gsm, bsm, msm, vsm):
    r2 = lambda p: p.reshape(1, -1)
    g0r, b0r, m0r, v0r = r2(gup0), r2(bup0), r2(mup0), r2(vup0)
    g1r, b1r, m1r, v1r = r2(gup1), r2(bup1), r2(mup1), r2(vup1)
    g2r, b2r, m2r, v2r = r2(gup2), r2(bup2), r2(mup2), r2(vup2)
    gsr, bsr, msr, vsr = r2(gsm), r2(bsm), r2(msm), r2(vsm)

    p1p = jnp.pad(parent1, (0, N1P - N1))
    o1p = jnp.pad(off1, (0, N1P - N1))
    p0p = jnp.pad(parent0, (0, N0P - N0))
    o0p = jnp.pad(off0, (0, N0P - N0))
    x1p = jnp.pad(x1, ((0, N1P - N1), (0, 0)))
    x0p = jnp.pad(x0, ((0, N0P - N0), (0, 0)))
    src2 = jnp.pad(edge_src, (0, EP - E)).reshape(_NCHT, _CH_SM)
    kid2 = jnp.pad(kidx, (0, EP - E)).reshape(_NCHT, _CH_SM)
    dst2 = jnp.pad(edge_dst, (0, EP - E),
                   constant_values=DUMMY_DST).reshape(_NCHT, _CH_SM)
    esk = jnp.stack([src2, kid2, dst2], axis=1)
    zrows = jnp.zeros((_ZCH, LD), jnp.float32)

    h2 = pl.pallas_call(
        _tc_a,
        out_shape=jax.ShapeDtypeStruct((8, N3, 128), jnp.float32),
    )(x3, Wup2, g2r, v2r)

    h1 = pl.pallas_call(
        _tc_b,
        out_shape=jax.ShapeDtypeStruct((8, N2, 256), jnp.float32),
    )(parent2.reshape(1, N2), off2.reshape(1, N2), h2.reshape(8 * N3, 128),
      g2r, b2r, m2r, v2r, x2, Wup1, g1r, v1r)
    up1 = _make_gather(8 * N2, 256, N2, 80, 1, 1)(h1.reshape(8 * N2, 256),
                                                  p1p, o1p)

    h0 = pl.pallas_call(
        _tc_c,
        grid=(8,),
        in_specs=[
            pl.BlockSpec((N1P, 256), lambda k: (0, 0)),
            pl.BlockSpec((1, 192), lambda k: (0, 0)),
            pl.BlockSpec((1, 192), lambda k: (0, 0)),
            pl.BlockSpec((1, 192), lambda k: (0, 0)),
            pl.BlockSpec((1, 192), lambda k: (0, 0)),
            pl.BlockSpec((N1P, 64), lambda k: (0, 0)),
            pl.BlockSpec((1, 256, 256), lambda k: (k, 0, 0)),
            pl.BlockSpec((1, 256), lambda k: (0, 0)),
            pl.BlockSpec((1, 256), lambda k: (0, 0)),
        ],
        out_specs=pl.BlockSpec((1, N1P, 256), lambda k: (k, 0, 0)),
        out_shape=jax.ShapeDtypeStruct((8, N1P, 256), jnp.float32),
    )(up1, g1r, b1r, m1r, v1r, x1p, Wup0, g0r, v0r)
    up0 = _make_gather(8 * N1P, 256, N1P, 80, 5, 3)(h0.reshape(8 * N1P, 256),
                                                    p0p, o0p)

    TB = 2048
    hs = pl.pallas_call(
        _tc_d,
        grid=(N0P // TB, 27),
        in_specs=[
            pl.BlockSpec((TB, 256), lambda t, k: (t, 0)),
            pl.BlockSpec((1, 256), lambda t, k: (0, 0)),
            pl.BlockSpec((1, 256), lambda t, k: (0, 0)),
            pl.BlockSpec((1, 256), lambda t, k: (0, 0)),
            pl.BlockSpec((1, 256), lambda t, k: (0, 0)),
            pl.BlockSpec((TB, 64), lambda t, k: (t, 0)),
            pl.BlockSpec((1, 320, 128), lambda t, k: (k, 0, 0)),
            pl.BlockSpec((1, 128), lambda t, k: (0, 0)),
            pl.BlockSpec((1, 128), lambda t, k: (0, 0)),
        ],
        out_specs=pl.BlockSpec((1, TB, 128), lambda t, k: (k, t, 0)),
        out_shape=jax.ShapeDtypeStruct((27, N0P, 128), jnp.float32),
    )(up0, g0r, b0r, m0r, v0r, x0p, Wsm, gsr, vsr)

    parts = _sc_smooth(hs.reshape(27 * N0P, 128), esk, zrows)

    out = pl.pallas_call(
        _tc_e,
        out_shape=jax.ShapeDtypeStruct((N0, LD), jnp.float32),
    )(parts, gsr, bsr, msr, vsr)
    return out


# speedup vs baseline: 1.7635x; 1.0216x over previous
import functools

import jax
import jax.numpy as jnp
from jax import lax
from jax.experimental import pallas as pl
from jax.experimental.pallas import tpu as pltpu
from jax.experimental.pallas import tpu_sc as plsc

EPS = 1e-5
NC, NS = 2, 16
NW = NC * NS

N0, N1, N2, N3 = 10000, 2500, 640, 160
N0P, N1P, N2G = 10240, 2560, 1024
E, EP = 160000, 163840
LD = 128
ACC_R = 10240
DUMMY_DST = 10200


def _rsqrt(v):
    return jax.lax.rsqrt(v + EPS)


def _tc_a(x3_ref, w_ref, g_ref, v_ref, o_ref):
    a = g_ref[...] * _rsqrt(v_ref[...])
    x = x3_ref[...]
    for k in range(8):
        o_ref[k] = jnp.dot(x, w_ref[k] * a, preferred_element_type=jnp.float32)


def _tc_b(p2_ref, o2_ref, h2_ref, g2, b2, m2, v2, x2_ref, w_ref, g1, v1,
          o_ref):
    flat = o2_ref[0] * N3 + p2_ref[0]
    oh = jnp.where(
        jax.lax.broadcasted_iota(jnp.int32, (N2, 8 * N3), 1)
        == flat.reshape(N2, 1), 1.0, 0.0).astype(jnp.float32)
    up2 = jnp.dot(oh, h2_ref[...], preferred_element_type=jnp.float32)
    a2 = g2[...] * _rsqrt(v2[...])
    c2 = b2[...] - m2[...] * a2
    u = jnp.maximum(up2 + c2, 0.0)
    r = jnp.concatenate([u, x2_ref[...]], axis=1)
    a1 = g1[...] * _rsqrt(v1[...])
    z = jnp.zeros((N2, 64), jnp.float32)
    for k in range(8):
        h = jnp.dot(r, w_ref[k] * a1, preferred_element_type=jnp.float32)
        o_ref[k] = jnp.concatenate([h, z], axis=1)


def _tc_c(up_ref, g1, b1, m1, v1, x1_ref, w_ref, g0, v0, o_ref):
    a1 = g1[...] * _rsqrt(v1[...])
    c1 = b1[...] - m1[...] * a1
    u = jnp.maximum(up_ref[:, 0:192] + c1, 0.0)
    r = jnp.concatenate([u, x1_ref[...]], axis=1).astype(jnp.bfloat16)
    a0 = g0[...] * _rsqrt(v0[...])
    w = (w_ref[0] * a0).astype(jnp.bfloat16)
    o_ref[0] = jnp.dot(r, w, preferred_element_type=jnp.float32)


def _tc_d(up_ref, g0, b0, m0, v0, x0_ref, w_ref, gs, vs, o_ref):
    a0 = g0[...] * _rsqrt(v0[...])
    c0 = b0[...] - m0[...] * a0
    u = jnp.maximum(up_ref[...] + c0, 0.0)
    r = jnp.concatenate([u, x0_ref[...]], axis=1).astype(jnp.bfloat16)
    asm = gs[...] * _rsqrt(vs[...])
    w = (w_ref[0] * asm).astype(jnp.bfloat16)
    o_ref[0] = jnp.dot(r, w, preferred_element_type=jnp.float32)


def _tc_e(p_ref, gs, bs, ms, vs, o_ref):
    asm = gs[...] * _rsqrt(vs[...])
    cs = bs[...] - ms[...] * asm
    s = p_ref[0] + p_ref[1]
    o_ref[...] = jnp.maximum(s[0:N0] + cs, 0.0)


def _make_gather(T, D, MULT, CH, ncha, nchb):
    NTOT = (ncha + nchb) * NS * CH
    nmax = max(ncha, nchb)
    mesh = plsc.VectorSubcoreMesh(core_axis_name="c", subcore_axis_name="s")

    @functools.partial(
        pl.kernel,
        out_type=jax.ShapeDtypeStruct((NTOT, D), jnp.float32),
        mesh=mesh,
        scratch_types=[
            pltpu.VMEM((nmax * CH,), jnp.int32),
            pltpu.VMEM((nmax * CH,), jnp.int32),
            pltpu.VMEM((nmax, CH), jnp.int32),
            [pltpu.VMEM((CH, D), jnp.float32) for _ in range(nmax)],
            [pltpu.SemaphoreType.DMA for _ in range(nmax)],
            [pltpu.SemaphoreType.DMA for _ in range(nmax)],
        ],
    )
    def g(table, parent, off, out, par_v, off_v, idx_v, rows, gsems, osems):
        cid = lax.axis_index("c")
        sid = lax.axis_index("s")

        def run(base, nch):
            nb = nch * CH
            pltpu.sync_copy(parent.at[pl.ds(base, nb)],
                            par_v.at[pl.ds(0, nb)])
            pltpu.sync_copy(off.at[pl.ds(base, nb)], off_v.at[pl.ds(0, nb)])
            for j in range(nch):
                for t in range(CH // 16):
                    s0 = j * CH + t * 16
                    idx_v[j, pl.ds(t * 16, 16)] = (
                        off_v[pl.ds(s0, 16)] * MULT + par_v[pl.ds(s0, 16)])
                pltpu.async_copy(table.at[idx_v.at[j]], rows[j], gsems[j])
            for j in range(nch):
                pltpu.make_async_copy(table.at[idx_v.at[j]], rows[j],
                                      gsems[j]).wait()
                pltpu.async_copy(rows[j], out.at[pl.ds(base + j * CH, CH)],
                                 osems[j])
            for j in range(nch):
                pltpu.make_async_copy(rows[j],
                                      out.at[pl.ds(base + j * CH, CH)],
                                      osems[j]).wait()

        @pl.when(cid == 0)
        def _():
            run(sid * (ncha * CH), ncha)

        @pl.when(cid != 0)
        def _():
            run(NS * (ncha * CH) + sid * (nchb * CH), nchb)

    return g


_CH_SM = 128
_NCHT = EP // _CH_SM
_CH_A = 68
_CH_B = _NCHT // NS - _CH_A
_ZCH = 128


def _sc_smooth(table, esk):
    mesh = plsc.VectorSubcoreMesh(core_axis_name="c", subcore_axis_name="s")
    rows_per_tile = ACC_R // NS

    @functools.partial(
        pl.kernel,
        out_type=jax.ShapeDtypeStruct((NC, ACC_R, LD), jnp.float32),
        mesh=mesh,
        scratch_types=[
            [pltpu.VMEM((1, 3, _CH_SM), jnp.int32) for _ in range(2)],
            [pltpu.VMEM((1, _CH_SM), jnp.int32) for _ in range(2)],
            [pltpu.VMEM((1, _CH_SM), jnp.int32) for _ in range(2)],
            [pltpu.VMEM((_CH_SM, LD), jnp.float32) for _ in range(2)],
            [pltpu.SemaphoreType.DMA for _ in range(2)],
            [pltpu.SemaphoreType.DMA for _ in range(2)],
            [pltpu.SemaphoreType.DMA for _ in range(2)],
            pltpu.VMEM_SHARED((ACC_R, LD), jnp.float32),
            pltpu.SemaphoreType.DMA,
        ],
    )
    def k(tab, ed, out, ebufs, gixs, dixs, rows, isems, gsems, ssems,
          acc, psem):
        cid = lax.axis_index("c")
        sid = lax.axis_index("s")
        start = jnp.where(cid == 0, sid * _CH_A, NS * _CH_A + sid * _CH_B)
        n2 = jnp.where(cid == 0, _CH_A // 2, _CH_B // 2)

        for t in range(2):
            pltpu.async_copy(ed.at[pl.ds(start + t, 1)], ebufs[t], isems[t])
        zv = jnp.zeros((16,), jnp.float32)

        def zfill(r, carry):
            for t in range(LD // 16):
                rows[0][r, pl.ds(t * 16, 16)] = zv
            return carry

        lax.fori_loop(0, _CH_SM, zfill, 0)
        nz = rows_per_tile // _ZCH
        for j in range(nz):
            pltpu.async_copy(
                rows[0],
                acc.at[pl.ds(sid * rows_per_tile + j * _ZCH, _ZCH)], psem)
        for j in range(nz):
            pltpu.make_async_copy(rows[0], acc.at[pl.ds(0, _ZCH)],
                                  psem).wait()
        plsc.subcore_barrier()

        def gbody(i, carry):
            for t in range(2):
                c = start + i * 2 + t
                pltpu.make_async_copy(ed.at[pl.ds(c, 1)], ebufs[t],
                                      isems[t]).wait()

                @pl.when(i > 0)
                def _():
                    pltpu.make_async_copy(rows[t], acc.at[dixs[t].at[0]],
                                          ssems[t]).wait()

                for v in range(_CH_SM // 16):
                    sl = pl.ds(v * 16, 16)
                    gixs[t][0, sl] = (ebufs[t][0, 1, sl] * N0P
                                      + ebufs[t][0, 0, sl])
                    dixs[t][0, sl] = ebufs[t][0, 2, sl]
                pltpu.async_copy(tab.at[gixs[t].at[0]], rows[t], gsems[t])
            for t in range(2):
                c = start + i * 2 + t
                pltpu.make_async_copy(tab.at[gixs[t].at[0]], rows[t],
                                      gsems[t]).wait()
                pltpu.async_copy(rows[t], acc.at[dixs[t].at[0]], ssems[t],
                                 add=True)

                @pl.when(i + 1 < n2)
                def _():
                    pltpu.async_copy(ed.at[pl.ds(c + 2, 1)], ebufs[t],
                                     isems[t])
            return carry

        lax.fori_loop(0, n2, gbody, 0)
        for t in range(2):
            pltpu.make_async_copy(rows[t], acc.at[dixs[t].at[0]],
                                  ssems[t]).wait()
        plsc.subcore_barrier()
        pltpu.sync_copy(acc.at[pl.ds(sid * rows_per_tile, rows_per_tile)],
                        out.at[cid, pl.ds(sid * rows_per_tile, rows_per_tile)])

    return k(table, esk)


def kernel(x0, x1, x2, x3, parent0, parent1, parent2, off0, off1, off2,
           edge_src, edge_dst, kidx,
           Wup0, gup0, bup0, mup0, vup0, Wup1, gup1, bup1, mup1, vup1,
           Wup2, gup2, bup2, mup2, vup2, Wsm, gsm, bsm, msm, vsm):
    r2 = lambda p: p.reshape(1, -1)
    g0r, b0r, m0r, v0r = r2(gup0), r2(bup0), r2(mup0), r2(vup0)
    g1r, b1r, m1r, v1r = r2(gup1), r2(bup1), r2(mup1), r2(vup1)
    g2r, b2r, m2r, v2r = r2(gup2), r2(bup2), r2(mup2), r2(vup2)
    gsr, bsr, msr, vsr = r2(gsm), r2(bsm), r2(msm), r2(vsm)

    p1p = jnp.pad(parent1, (0, N1P - N1))
    o1p = jnp.pad(off1, (0, N1P - N1))
    p0p = jnp.pad(parent0, (0, N0P - N0))
    o0p = jnp.pad(off0, (0, N0P - N0))
    x1p = jnp.pad(x1, ((0, N1P - N1), (0, 0)))
    x0p = jnp.pad(x0, ((0, N0P - N0), (0, 0)))
    src2 = jnp.pad(edge_src, (0, EP - E)).reshape(_NCHT, _CH_SM)
    kid2 = jnp.pad(kidx, (0, EP - E)).reshape(_NCHT, _CH_SM)
    dst2 = jnp.pad(edge_dst, (0, EP - E),
                   constant_values=DUMMY_DST).reshape(_NCHT, _CH_SM)
    esk = jnp.stack([src2, kid2, dst2], axis=1)

    h2 = pl.pallas_call(
        _tc_a,
        out_shape=jax.ShapeDtypeStruct((8, N3, 128), jnp.float32),
    )(x3, Wup2, g2r, v2r)

    h1 = pl.pallas_call(
        _tc_b,
        out_shape=jax.ShapeDtypeStruct((8, N2, 256), jnp.float32),
    )(parent2.reshape(1, N2), off2.reshape(1, N2), h2.reshape(8 * N3, 128),
      g2r, b2r, m2r, v2r, x2, Wup1, g1r, v1r)
    up1 = _make_gather(8 * N2, 256, N2, 80, 1, 1)(h1.reshape(8 * N2, 256),
                                                  p1p, o1p)

    h0 = pl.pallas_call(
        _tc_c,
        grid=(8,),
        in_specs=[
            pl.BlockSpec((N1P, 256), lambda k: (0, 0)),
            pl.BlockSpec((1, 192), lambda k: (0, 0)),
            pl.BlockSpec((1, 192), lambda k: (0, 0)),
            pl.BlockSpec((1, 192), lambda k: (0, 0)),
            pl.BlockSpec((1, 192), lambda k: (0, 0)),
            pl.BlockSpec((N1P, 64), lambda k: (0, 0)),
            pl.BlockSpec((1, 256, 256), lambda k: (k, 0, 0)),
            pl.BlockSpec((1, 256), lambda k: (0, 0)),
            pl.BlockSpec((1, 256), lambda k: (0, 0)),
        ],
        out_specs=pl.BlockSpec((1, N1P, 256), lambda k: (k, 0, 0)),
        out_shape=jax.ShapeDtypeStruct((8, N1P, 256), jnp.float32),
    )(up1, g1r, b1r, m1r, v1r, x1p, Wup0, g0r, v0r)
    up0 = _make_gather(8 * N1P, 256, N1P, 80, 5, 3)(h0.reshape(8 * N1P, 256),
                                                    p0p, o0p)

    TB = 2048
    hs = pl.pallas_call(
        _tc_d,
        grid=(N0P // TB, 27),
        in_specs=[
            pl.BlockSpec((TB, 256), lambda t, k: (t, 0)),
            pl.BlockSpec((1, 256), lambda t, k: (0, 0)),
            pl.BlockSpec((1, 256), lambda t, k: (0, 0)),
            pl.BlockSpec((1, 256), lambda t, k: (0, 0)),
            pl.BlockSpec((1, 256), lambda t, k: (0, 0)),
            pl.BlockSpec((TB, 64), lambda t, k: (t, 0)),
            pl.BlockSpec((1, 320, 128), lambda t, k: (k, 0, 0)),
            pl.BlockSpec((1, 128), lambda t, k: (0, 0)),
            pl.BlockSpec((1, 128), lambda t, k: (0, 0)),
        ],
        out_specs=pl.BlockSpec((1, TB, 128), lambda t, k: (k, t, 0)),
        out_shape=jax.ShapeDtypeStruct((27, N0P, 128), jnp.float32),
    )(up0, g0r, b0r, m0r, v0r, x0p, Wsm, gsr, vsr)

    parts = _sc_smooth(hs.reshape(27 * N0P, 128), esk)

    out = pl.pallas_call(
        _tc_e,
        out_shape=jax.ShapeDtypeStruct((N0, LD), jnp.float32),
    )(parts, gsr, bsr, msr, vsr)
    return out
